# baseline (speedup 1.0000x reference)
import jax
import jax.numpy as jnp
from jax import lax
from jax.experimental import pallas as pl
from jax.experimental.pallas import tpu as pltpu
from jax.experimental.pallas import tpu_sc as plsc

N = 10000
NPAD = 10240
NNZ = 320000
D_IN = 128
D_HID = 128
D_OUT = 64
NC = 2
NS = 16
NW = NC * NS
CHUNK = 125
NROW = NNZ // CHUNK
RPS = NPAD // NS
BB = 128
CW = 16

NCHUNKF = NNZ // NS // CHUNK
NPHF = 8
HCHF = NCHUNKF // NPHF

NCHUNKC = NNZ // NW // CHUNK

_MESH = plsc.VectorSubcoreMesh(core_axis_name="c", subcore_axis_name="s")
_SC_PARAMS = pltpu.CompilerParams(use_tc_tiling_on_sc=False)


def _sc_layer(tsplit, src2, dst2, binvw, zeros_h):
    Dh = tsplit.shape[2]

    def body(t_ref, src_ref, dst_ref, bw_ref, zero_ref,
             hpre_ref, etmp_ref,
             sidx, didx, rows0, rows1, bblk, acc, sem0, sem1, zsem):
        c = lax.axis_index("c")
        s = lax.axis_index("s")
        r0 = s * RPS
        pltpu.async_copy(zero_ref.at[pl.ds(r0, RPS)],
                         acc.at[pl.ds(r0, RPS)], zsem).wait()
        plsc.subcore_barrier()

        def sweep(table, gs_ref, gd_ref):
            for p in range(NPHF):
                base = s * NCHUNKF + p * HCHF
                pltpu.sync_copy(gs_ref.at[pl.ds(base, HCHF)], sidx)
                pltpu.sync_copy(gd_ref.at[pl.ds(base, HCHF)], didx)
                pltpu.async_copy(table.at[sidx.at[0]],
                                 rows0.at[pl.ds(0, CHUNK)], sem0)

                @pl.loop(0, HCHF // 2)
                def _(i):
                    j0 = 2 * i
                    pltpu.async_copy(table.at[sidx.at[j0 + 1]],
                                     rows1.at[pl.ds(0, CHUNK)], sem1)
                    pltpu.make_async_copy(table.at[sidx.at[j0]],
                                          rows0.at[pl.ds(0, CHUNK)],
                                          sem0).wait()
                    pltpu.sync_copy(rows0.at[pl.ds(0, CHUNK)],
                                    acc.at[didx.at[j0]], add=True)

                    @pl.when(j0 + 2 < HCHF)
                    def _():
                        pltpu.async_copy(table.at[sidx.at[j0 + 2]],
                                         rows0.at[pl.ds(0, CHUNK)], sem0)

                    pltpu.make_async_copy(table.at[sidx.at[j0 + 1]],
                                          rows1.at[pl.ds(0, CHUNK)],
                                          sem1).wait()
                    pltpu.sync_copy(rows1.at[pl.ds(0, CHUNK)],
                                    acc.at[didx.at[j0 + 1]], add=True)

        sweep(t_ref.at[c], src_ref, dst_ref)
        plsc.subcore_barrier()

        @pl.loop(0, RPS // BB)
        def _(b):
            off = r0 + b * BB
            pltpu.sync_copy(acc.at[pl.ds(off, BB)], rows0)
            pltpu.sync_copy(bw_ref.at[pl.ds(off, BB)], bblk)

            @pl.loop(0, BB)
            def _(r):
                for k in range(Dh // 16):
                    sl = pl.ds(k * 16, 16)
                    rows0.at[r][sl] = rows0.at[r][sl] * bblk.at[r][sl]

            pltpu.sync_copy(rows0, etmp_ref.at[c, pl.ds(off, BB)])
            pltpu.sync_copy(zero_ref.at[pl.ds(off, BB)],
                            acc.at[pl.ds(off, BB)])

        plsc.subcore_barrier()
        sweep(etmp_ref.at[c], dst_ref, src_ref)
        plsc.subcore_barrier()
        pltpu.sync_copy(acc.at[pl.ds(r0, RPS)],
                        hpre_ref.at[pl.ds(r0, RPS), pl.ds(c * Dh, Dh)])

    return pl.kernel(
        body,
        out_type=(jax.ShapeDtypeStruct((NPAD, 2 * Dh), jnp.float32),
                  jax.ShapeDtypeStruct((NC, NPAD, Dh), jnp.float32)),
        mesh=_MESH,
        compiler_params=_SC_PARAMS,
        scratch_types=[
            pltpu.VMEM((HCHF, CHUNK), jnp.int32),
            pltpu.VMEM((HCHF, CHUNK), jnp.int32),
            pltpu.VMEM((BB, Dh), jnp.float32),
            pltpu.VMEM((BB, Dh), jnp.float32),
            pltpu.VMEM((BB, Dh), jnp.float32),
            pltpu.VMEM_SHARED((NPAD, Dh), jnp.float32),
            pltpu.SemaphoreType.DMA,
            pltpu.SemaphoreType.DMA,
            pltpu.SemaphoreType.DMA,
        ],
    )(tsplit, src2, dst2, binvw, zeros_h)


def _sc_counts(src2, dst2, ones_c, zeros_c):
    def body(src_ref, dst_ref, ones_ref, zc_ref,
             dcnt_ref, bcnt_ref, sidx, didx, ones_v, dacc, bacc):
        c = lax.axis_index("c")
        s = lax.axis_index("s")
        w = c * NS + s
        r0 = s * RPS
        pltpu.sync_copy(zc_ref.at[pl.ds(r0, RPS)], dacc.at[pl.ds(r0, RPS)])
        pltpu.sync_copy(zc_ref.at[pl.ds(r0, RPS)], bacc.at[pl.ds(r0, RPS)])
        pltpu.sync_copy(ones_ref, ones_v)
        pltpu.sync_copy(src_ref.at[pl.ds(w * NCHUNKC, NCHUNKC)], sidx)
        pltpu.sync_copy(dst_ref.at[pl.ds(w * NCHUNKC, NCHUNKC)], didx)
        plsc.subcore_barrier()

        @pl.loop(0, NCHUNKC)
        def _(j):
            pltpu.sync_copy(ones_v, dacc.at[sidx.at[j]], add=True)
            pltpu.sync_copy(ones_v, bacc.at[didx.at[j]], add=True)

        plsc.subcore_barrier()
        sl = pl.ds(r0, RPS)
        pltpu.sync_copy(dacc.at[sl], dcnt_ref.at[c, sl])
        pltpu.sync_copy(bacc.at[sl], bcnt_ref.at[c, sl])

    return pl.kernel(
        body,
        out_type=(jax.ShapeDtypeStruct((NC, NPAD, CW), jnp.float32),
                  jax.ShapeDtypeStruct((NC, NPAD, CW), jnp.float32)),
        mesh=_MESH,
        compiler_params=_SC_PARAMS,
        scratch_types=[
            pltpu.VMEM((NCHUNKC, CHUNK), jnp.int32),
            pltpu.VMEM((NCHUNKC, CHUNK), jnp.int32),
            pltpu.VMEM((CHUNK, CW), jnp.float32),
            pltpu.VMEM_SHARED((NPAD, CW), jnp.float32),
            pltpu.VMEM_SHARED((NPAD, CW), jnp.float32),
        ],
    )(src2, dst2, ones_c, zeros_c)


def _inv_col(cnt_ref):
    cnt = cnt_ref[0, :, 0:1] + cnt_ref[1, :, 0:1]
    return jnp.where(cnt > 0, 1.0 / jnp.where(cnt > 0, cnt, 1.0), 0.0)


def _tc0_body(x_ref, w_ref, bcnt_ref, t_ref, bw_ref):
    xw = jnp.dot(x_ref[...], w_ref[...], preferred_element_type=jnp.float32)
    t_ref[0] = xw[:, :D_HID // 2]
    t_ref[1] = xw[:, D_HID // 2:]
    bw_ref[...] = jnp.broadcast_to(_inv_col(bcnt_ref), (NPAD, D_HID // 2))


def _tc0(x, wT, bcnt):
    return pl.pallas_call(
        _tc0_body,
        out_shape=(jax.ShapeDtypeStruct((NC, N, D_HID // 2), jnp.float32),
                   jax.ShapeDtypeStruct((NPAD, D_HID // 2), jnp.float32)),
    )(x, wT, bcnt)


def _tc1_body(hpre_ref, dcnt_ref, bcnt_ref, b_ref, w_ref,
              h_ref, t_ref, bw_ref):
    h = jnp.maximum(hpre_ref[...] * _inv_col(dcnt_ref) + b_ref[...], 0.0)
    h_ref[...] = h
    xw = jnp.dot(h, w_ref[...], preferred_element_type=jnp.float32)
    t_ref[0] = xw[:, :D_OUT // 2]
    t_ref[1] = xw[:, D_OUT // 2:]
    bw_ref[...] = jnp.broadcast_to(_inv_col(bcnt_ref), (NPAD, D_OUT // 2))


def _tc1(hpre, dcnt, bcnt, b, wT):
    return pl.pallas_call(
        _tc1_body,
        out_shape=(jax.ShapeDtypeStruct((NPAD, D_HID), jnp.float32),
                   jax.ShapeDtypeStruct((NC, NPAD, D_OUT // 2), jnp.float32),
                   jax.ShapeDtypeStruct((NPAD, D_OUT // 2), jnp.float32)),
    )(hpre, dcnt, bcnt, b, wT)


def _tc2_body(hpre_ref, dcnt_ref, b_ref, w_ref, bp_ref, h_ref, z_ref):
    h = jnp.maximum(hpre_ref[...] * _inv_col(dcnt_ref) + b_ref[...], 0.0)
    h_ref[...] = h
    z_ref[...] = jnp.dot(h, w_ref[...],
                         preferred_element_type=jnp.float32) + bp_ref[...]


def _tc2(hpre, dcnt, b, wT, bp):
    return pl.pallas_call(
        _tc2_body,
        out_shape=(jax.ShapeDtypeStruct((NPAD, D_OUT), jnp.float32),
                   jax.ShapeDtypeStruct((NPAD, D_OUT), jnp.float32)),
    )(hpre, dcnt, b, wT, bp)


def kernel(x, hyperedge_index, W1, b1, W2, b2, Wp, bp):
    node2 = hyperedge_index[0].reshape(NROW, CHUNK)
    edge2 = hyperedge_index[1].reshape(NROW, CHUNK)
    z64 = jnp.zeros((NPAD, D_HID // 2), jnp.float32)
    z32 = jnp.zeros((NPAD, D_OUT // 2), jnp.float32)
    zc = jnp.zeros((NPAD, CW), jnp.float32)
    ones_c = jnp.ones((CHUNK, CW), jnp.float32)

    dcnt, bcnt = _sc_counts(node2, edge2, ones_c, zc)
    tsplit, bw64 = _tc0(x, W1.T, bcnt)
    h1pre, _ = _sc_layer(tsplit, node2, edge2, bw64, z64)
    h1, t2, bw32 = _tc1(h1pre, dcnt, bcnt, b1.reshape(1, -1), W2.T)
    h2pre, _ = _sc_layer(t2, node2, edge2, bw32, z32)
    h2, z = _tc2(h2pre, dcnt, b2.reshape(1, -1), Wp.T, bp.reshape(1, -1))
    return (z[:N], h1[:N], h2[:N])


# speedup vs baseline: 1.2608x; 1.2608x over previous
import jax
import jax.numpy as jnp
from jax import lax
from jax.experimental import pallas as pl
from jax.experimental.pallas import tpu as pltpu
from jax.experimental.pallas import tpu_sc as plsc

N = 10000
NPAD = 10240
NNZ = 320000
D_IN = 128
D_HID = 128
D_OUT = 64
NC = 2
NS = 16
NW = NC * NS
PER_W = NNZ // NW
CHUNK = 125
NCHUNK = PER_W // CHUNK
NPH = 2
HCH = NCHUNK // NPH
RPS = NPAD // NS
CW = 16

_MESH = plsc.VectorSubcoreMesh(core_axis_name="c", subcore_axis_name="s")


def _sc_pass(table, src2, dst2, zeros_nd):
    D = table.shape[1]

    def body(table_ref, src_ref, dst_ref, zero_ref,
             out_ref, sidx, didx, rows0, rows1, acc, sem0, sem1, zsem):
        c = lax.axis_index("c")
        s = lax.axis_index("s")
        w = c * NS + s
        r0 = s * RPS
        zcp = pltpu.async_copy(zero_ref.at[pl.ds(r0, RPS)],
                               acc.at[pl.ds(r0, RPS)], zsem)
        zcp.wait()
        plsc.subcore_barrier()

        for p in range(NPH):
            base = w * NCHUNK + p * HCH
            pltpu.sync_copy(src_ref.at[pl.ds(base, HCH)], sidx)
            pltpu.sync_copy(dst_ref.at[pl.ds(base, HCH)], didx)
            pltpu.async_copy(table_ref.at[sidx.at[0]], rows0, sem0)

            @pl.loop(0, HCH // 2)
            def _(i):
                j0 = 2 * i
                pltpu.async_copy(table_ref.at[sidx.at[j0 + 1]], rows1, sem1)
                pltpu.make_async_copy(table_ref.at[sidx.at[j0]], rows0,
                                      sem0).wait()
                pltpu.sync_copy(rows0, acc.at[didx.at[j0]], add=True)

                @pl.when(j0 + 2 < HCH)
                def _():
                    pltpu.async_copy(table_ref.at[sidx.at[j0 + 2]], rows0, sem0)

                pltpu.make_async_copy(table_ref.at[sidx.at[j0 + 1]], rows1,
                                      sem1).wait()
                pltpu.sync_copy(rows1, acc.at[didx.at[j0 + 1]], add=True)

        plsc.subcore_barrier()
        sl = pl.ds(r0, RPS)
        pltpu.sync_copy(acc.at[sl], out_ref.at[c, sl])

    return pl.kernel(
        body,
        out_type=jax.ShapeDtypeStruct((NC, NPAD, D), jnp.float32),
        mesh=_MESH,
        compiler_params=(pltpu.CompilerParams(use_tc_tiling_on_sc=False)
                         if D % 128 else None),
        scratch_types=[
            pltpu.VMEM((HCH, CHUNK), jnp.int32),
            pltpu.VMEM((HCH, CHUNK), jnp.int32),
            pltpu.VMEM((CHUNK, D), jnp.float32),
            pltpu.VMEM((CHUNK, D), jnp.float32),
            pltpu.VMEM_SHARED((NPAD, D), jnp.float32),
            pltpu.SemaphoreType.DMA,
            pltpu.SemaphoreType.DMA,
            pltpu.SemaphoreType.DMA,
        ],
    )(table, src2, dst2, zeros_nd)


def _sc_counts(src2, dst2, ones_c, zeros_c):
    def body(src_ref, dst_ref, ones_ref, zc_ref,
             dcnt_ref, bcnt_ref, sidx, didx, ones_v, dacc, bacc):
        c = lax.axis_index("c")
        s = lax.axis_index("s")
        w = c * NS + s
        r0 = s * RPS
        pltpu.sync_copy(zc_ref.at[pl.ds(r0, RPS)], dacc.at[pl.ds(r0, RPS)])
        pltpu.sync_copy(zc_ref.at[pl.ds(r0, RPS)], bacc.at[pl.ds(r0, RPS)])
        pltpu.sync_copy(ones_ref, ones_v)
        pltpu.sync_copy(src_ref.at[pl.ds(w * NCHUNK, NCHUNK)], sidx)
        pltpu.sync_copy(dst_ref.at[pl.ds(w * NCHUNK, NCHUNK)], didx)
        plsc.subcore_barrier()

        @pl.loop(0, NCHUNK)
        def _(j):
            pltpu.sync_copy(ones_v, dacc.at[sidx.at[j]], add=True)
            pltpu.sync_copy(ones_v, bacc.at[didx.at[j]], add=True)

        plsc.subcore_barrier()
        sl = pl.ds(r0, RPS)
        pltpu.sync_copy(dacc.at[sl], dcnt_ref.at[c, sl])
        pltpu.sync_copy(bacc.at[sl], bcnt_ref.at[c, sl])

    return pl.kernel(
        body,
        out_type=(jax.ShapeDtypeStruct((NC, NPAD, CW), jnp.float32),
                  jax.ShapeDtypeStruct((NC, NPAD, CW), jnp.float32)),
        mesh=_MESH,
        compiler_params=pltpu.CompilerParams(use_tc_tiling_on_sc=False),
        scratch_types=[
            pltpu.VMEM((NCHUNK, CHUNK), jnp.int32),
            pltpu.VMEM((NCHUNK, CHUNK), jnp.int32),
            pltpu.VMEM((CHUNK, CW), jnp.float32),
            pltpu.VMEM_SHARED((NPAD, CW), jnp.float32),
            pltpu.VMEM_SHARED((NPAD, CW), jnp.float32),
        ],
    )(src2, dst2, ones_c, zeros_c)


def _inv_from(cnt_ref):
    cnt = cnt_ref[0, :, 0:1] + cnt_ref[1, :, 0:1]
    return jnp.where(cnt > 0, 1.0 / jnp.where(cnt > 0, cnt, 1.0), 0.0)


def _mm_body(x_ref, w_ref, o_ref):
    o_ref[...] = jnp.dot(x_ref[...], w_ref[...],
                         preferred_element_type=jnp.float32)


def _mm(x, wT):
    return pl.pallas_call(
        _mm_body,
        out_shape=jax.ShapeDtypeStruct((x.shape[0], wT.shape[1]), jnp.float32),
    )(x, wT)


def _combine_body(p_ref, cnt_ref, o_ref):
    o_ref[...] = (p_ref[0] + p_ref[1]) * _inv_from(cnt_ref)


def _combine(p, cnt):
    return pl.pallas_call(
        _combine_body,
        out_shape=jax.ShapeDtypeStruct(p.shape[1:], jnp.float32),
    )(p, cnt)


def _layer_body(p_ref, cnt_ref, b_ref, w_ref, h_ref, xw_ref):
    h = jnp.maximum((p_ref[0] + p_ref[1]) * _inv_from(cnt_ref) + b_ref[...],
                    0.0)[:N]
    h_ref[...] = h
    xw_ref[...] = jnp.dot(h, w_ref[...], preferred_element_type=jnp.float32)


def _layer(p, cnt, b, wT):
    d = p.shape[2]
    return pl.pallas_call(
        _layer_body,
        out_shape=(jax.ShapeDtypeStruct((N, d), jnp.float32),
                   jax.ShapeDtypeStruct((N, wT.shape[1]), jnp.float32)),
    )(p, cnt, b, wT)


def _final_body(p_ref, cnt_ref, b_ref, w_ref, bp_ref, h_ref, z_ref):
    h = jnp.maximum((p_ref[0] + p_ref[1]) * _inv_from(cnt_ref) + b_ref[...],
                    0.0)[:N]
    h_ref[...] = h
    z_ref[...] = jnp.dot(h, w_ref[...],
                         preferred_element_type=jnp.float32) + bp_ref[...]


def _final(p, cnt, b, wT, bp):
    d = p.shape[2]
    return pl.pallas_call(
        _final_body,
        out_shape=(jax.ShapeDtypeStruct((N, d), jnp.float32),
                   jax.ShapeDtypeStruct((N, wT.shape[1]), jnp.float32)),
    )(p, cnt, b, wT, bp)


def kernel(x, hyperedge_index, W1, b1, W2, b2, Wp, bp):
    node2 = hyperedge_index[0].reshape(NW * NCHUNK, CHUNK)
    edge2 = hyperedge_index[1].reshape(NW * NCHUNK, CHUNK)
    z128 = jnp.zeros((NPAD, D_HID), jnp.float32)
    z64 = jnp.zeros((NPAD, D_OUT), jnp.float32)
    zc = jnp.zeros((NPAD, CW), jnp.float32)
    ones_c = jnp.ones((CHUNK, CW), jnp.float32)

    dcnt, bcnt = _sc_counts(node2, edge2, ones_c, zc)
    xW1 = _mm(x, W1.T)
    e1p = _sc_pass(xW1, node2, edge2, z128)
    e1 = _combine(e1p, bcnt)
    h1p = _sc_pass(e1, edge2, node2, z128)
    h1f, xW2 = _layer(h1p, dcnt, b1.reshape(1, -1), W2.T)
    e2p = _sc_pass(xW2, node2, edge2, z64)
    e2 = _combine(e2p, bcnt)
    h2p = _sc_pass(e2, edge2, node2, z64)
    h2f, zf = _final(h2p, dcnt, b2.reshape(1, -1), Wp.T, bp.reshape(1, -1))
    return (zf, h1f, h2f)


# speedup vs baseline: 1.2790x; 1.0145x over previous
import jax
import jax.numpy as jnp
from jax import lax
from jax.experimental import pallas as pl
from jax.experimental.pallas import tpu as pltpu
from jax.experimental.pallas import tpu_sc as plsc

N = 10000
NPAD = 10240
NNZ = 320000
D_IN = 128
D_HID = 128
D_OUT = 64
NC = 2
NS = 16
NW = NC * NS
PER_W = NNZ // NW
CHUNK = 125
NCHUNK = PER_W // CHUNK
NPH = 2
HCH = NCHUNK // NPH
RPS = NPAD // NS
CW = 8

_MESH = plsc.VectorSubcoreMesh(core_axis_name="c", subcore_axis_name="s")


def _sc_pass(table, src2, dst2, zeros_nd):
    D = table.shape[1]

    def body(table_ref, src_ref, dst_ref, zero_ref,
             out_ref, sidx, didx, rows0, rows1, acc, sem0, sem1, zsem):
        c = lax.axis_index("c")
        s = lax.axis_index("s")
        w = c * NS + s
        r0 = s * RPS
        zcp = pltpu.async_copy(zero_ref.at[pl.ds(r0, RPS)],
                               acc.at[pl.ds(r0, RPS)], zsem)
        zcp.wait()
        plsc.subcore_barrier()

        for p in range(NPH):
            base = w * NCHUNK + p * HCH
            pltpu.sync_copy(src_ref.at[pl.ds(base, HCH)], sidx)
            pltpu.sync_copy(dst_ref.at[pl.ds(base, HCH)], didx)
            pltpu.async_copy(table_ref.at[sidx.at[0]], rows0, sem0)

            @pl.loop(0, HCH // 2)
            def _(i):
                j0 = 2 * i
                pltpu.async_copy(table_ref.at[sidx.at[j0 + 1]], rows1, sem1)
                pltpu.make_async_copy(table_ref.at[sidx.at[j0]], rows0,
                                      sem0).wait()
                pltpu.sync_copy(rows0, acc.at[didx.at[j0]], add=True)

                @pl.when(j0 + 2 < HCH)
                def _():
                    pltpu.async_copy(table_ref.at[sidx.at[j0 + 2]], rows0, sem0)

                pltpu.make_async_copy(table_ref.at[sidx.at[j0 + 1]], rows1,
                                      sem1).wait()
                pltpu.sync_copy(rows1, acc.at[didx.at[j0 + 1]], add=True)

        plsc.subcore_barrier()
        sl = pl.ds(r0, RPS)
        pltpu.sync_copy(acc.at[sl], out_ref.at[c, sl])

    return pl.kernel(
        body,
        out_type=jax.ShapeDtypeStruct((NC, NPAD, D), jnp.float32),
        mesh=_MESH,
        compiler_params=(pltpu.CompilerParams(use_tc_tiling_on_sc=False)
                         if D % 128 else None),
        scratch_types=[
            pltpu.VMEM((HCH, CHUNK), jnp.int32),
            pltpu.VMEM((HCH, CHUNK), jnp.int32),
            pltpu.VMEM((CHUNK, D), jnp.float32),
            pltpu.VMEM((CHUNK, D), jnp.float32),
            pltpu.VMEM_SHARED((NPAD, D), jnp.float32),
            pltpu.SemaphoreType.DMA,
            pltpu.SemaphoreType.DMA,
            pltpu.SemaphoreType.DMA,
        ],
    )(table, src2, dst2, zeros_nd)


def _sc_counts(src2, dst2, ones_c, zeros_c):
    def body(src_ref, dst_ref, ones_ref, zc_ref,
             dcnt_ref, bcnt_ref, sidx, didx, ones_v, dacc, bacc):
        c = lax.axis_index("c")
        s = lax.axis_index("s")
        w = c * NS + s
        r0 = s * RPS
        pltpu.sync_copy(zc_ref.at[pl.ds(r0, RPS)], dacc.at[pl.ds(r0, RPS)])
        pltpu.sync_copy(zc_ref.at[pl.ds(r0, RPS)], bacc.at[pl.ds(r0, RPS)])
        pltpu.sync_copy(ones_ref, ones_v)
        pltpu.sync_copy(src_ref.at[pl.ds(w * NCHUNK, NCHUNK)], sidx)
        pltpu.sync_copy(dst_ref.at[pl.ds(w * NCHUNK, NCHUNK)], didx)
        plsc.subcore_barrier()

        @pl.loop(0, NCHUNK)
        def _(j):
            pltpu.sync_copy(ones_v, dacc.at[sidx.at[j]], add=True)
            pltpu.sync_copy(ones_v, bacc.at[didx.at[j]], add=True)

        plsc.subcore_barrier()
        sl = pl.ds(r0, RPS)
        pltpu.sync_copy(dacc.at[sl], dcnt_ref.at[c, sl])
        pltpu.sync_copy(bacc.at[sl], bcnt_ref.at[c, sl])

    return pl.kernel(
        body,
        out_type=(jax.ShapeDtypeStruct((NC, NPAD, CW), jnp.float32),
                  jax.ShapeDtypeStruct((NC, NPAD, CW), jnp.float32)),
        mesh=_MESH,
        compiler_params=pltpu.CompilerParams(use_tc_tiling_on_sc=False),
        scratch_types=[
            pltpu.VMEM((NCHUNK, CHUNK), jnp.int32),
            pltpu.VMEM((NCHUNK, CHUNK), jnp.int32),
            pltpu.VMEM((CHUNK, CW), jnp.float32),
            pltpu.VMEM_SHARED((NPAD, CW), jnp.float32),
            pltpu.VMEM_SHARED((NPAD, CW), jnp.float32),
        ],
    )(src2, dst2, ones_c, zeros_c)


def _inv_from(cnt_ref):
    cnt = cnt_ref[0, :, 0:1] + cnt_ref[1, :, 0:1]
    return jnp.where(cnt > 0, 1.0 / jnp.where(cnt > 0, cnt, 1.0), 0.0)


def _mm_body(x_ref, w_ref, o_ref):
    o_ref[...] = jnp.dot(x_ref[...], w_ref[...],
                         preferred_element_type=jnp.float32)


def _mm(x, wT):
    return pl.pallas_call(
        _mm_body,
        out_shape=jax.ShapeDtypeStruct((x.shape[0], wT.shape[1]), jnp.float32),
    )(x, wT)


def _combine_body(p_ref, cnt_ref, o_ref):
    o_ref[...] = (p_ref[0] + p_ref[1]) * _inv_from(cnt_ref)


def _combine(p, cnt):
    return pl.pallas_call(
        _combine_body,
        out_shape=jax.ShapeDtypeStruct(p.shape[1:], jnp.float32),
    )(p, cnt)


def _layer_body(p_ref, cnt_ref, b_ref, w_ref, h_ref, xw_ref):
    h = jnp.maximum((p_ref[0] + p_ref[1]) * _inv_from(cnt_ref) + b_ref[...],
                    0.0)[:N]
    h_ref[...] = h
    xw_ref[...] = jnp.dot(h, w_ref[...], preferred_element_type=jnp.float32)


def _layer(p, cnt, b, wT):
    d = p.shape[2]
    return pl.pallas_call(
        _layer_body,
        out_shape=(jax.ShapeDtypeStruct((N, d), jnp.float32),
                   jax.ShapeDtypeStruct((N, wT.shape[1]), jnp.float32)),
    )(p, cnt, b, wT)


def _final_body(p_ref, cnt_ref, b_ref, w_ref, bp_ref, h_ref, z_ref):
    h = jnp.maximum((p_ref[0] + p_ref[1]) * _inv_from(cnt_ref) + b_ref[...],
                    0.0)[:N]
    h_ref[...] = h
    z_ref[...] = jnp.dot(h, w_ref[...],
                         preferred_element_type=jnp.float32) + bp_ref[...]


def _final(p, cnt, b, wT, bp):
    d = p.shape[2]
    return pl.pallas_call(
        _final_body,
        out_shape=(jax.ShapeDtypeStruct((N, d), jnp.float32),
                   jax.ShapeDtypeStruct((N, wT.shape[1]), jnp.float32)),
    )(p, cnt, b, wT, bp)


def kernel(x, hyperedge_index, W1, b1, W2, b2, Wp, bp):
    node2 = hyperedge_index[0].reshape(NW * NCHUNK, CHUNK)
    edge2 = hyperedge_index[1].reshape(NW * NCHUNK, CHUNK)
    z128 = jnp.zeros((NPAD, D_HID), jnp.float32)
    z64 = jnp.zeros((NPAD, D_OUT), jnp.float32)
    zc = jnp.zeros((NPAD, CW), jnp.float32)
    ones_c = jnp.ones((CHUNK, CW), jnp.float32)

    dcnt, bcnt = _sc_counts(node2, edge2, ones_c, zc)
    xW1 = _mm(x, W1.T)
    e1p = _sc_pass(xW1, node2, edge2, z128)
    e1 = _combine(e1p, bcnt)
    h1p = _sc_pass(e1, edge2, node2, z128)
    h1f, xW2 = _layer(h1p, dcnt, b1.reshape(1, -1), W2.T)
    e2p = _sc_pass(xW2, node2, edge2, z64)
    e2 = _combine(e2p, bcnt)
    h2p = _sc_pass(e2, edge2, node2, z64)
    h2f, zf = _final(h2p, dcnt, b2.reshape(1, -1), Wp.T, bp.reshape(1, -1))
    return (zf, h1f, h2f)


# speedup vs baseline: 1.2812x; 1.0017x over previous
import jax
import jax.numpy as jnp
from jax import lax
from jax.experimental import pallas as pl
from jax.experimental.pallas import tpu as pltpu
from jax.experimental.pallas import tpu_sc as plsc

N = 10000
NPAD = 10240
NNZ = 320000
D_IN = 128
D_HID = 128
D_OUT = 64
NC = 2
NS = 16
NW = NC * NS
NNZP = 327680
PER_W = NNZP // NW
CHUNK = 128
NCHUNK = PER_W // CHUNK
NPH = 2
HCH = NCHUNK // NPH
RPS = NPAD // NS
CW = 8

_MESH = plsc.VectorSubcoreMesh(core_axis_name="c", subcore_axis_name="s")


def _sc_pass(table, src2, dst2, zeros_nd):
    D = table.shape[1]

    def body(table_ref, src_ref, dst_ref, zero_ref,
             out_ref, sidx, didx, rows0, rows1, acc, sem0, sem1, zsem):
        c = lax.axis_index("c")
        s = lax.axis_index("s")
        w = c * NS + s
        r0 = s * RPS
        zcp = pltpu.async_copy(zero_ref.at[pl.ds(r0, RPS)],
                               acc.at[pl.ds(r0, RPS)], zsem)
        zcp.wait()
        plsc.subcore_barrier()

        for p in range(NPH):
            base = w * NCHUNK + p * HCH
            pltpu.sync_copy(src_ref.at[pl.ds(base, HCH)], sidx)
            pltpu.sync_copy(dst_ref.at[pl.ds(base, HCH)], didx)
            pltpu.async_copy(table_ref.at[sidx.at[0]], rows0, sem0)

            @pl.loop(0, HCH // 2)
            def _(i):
                j0 = 2 * i
                pltpu.async_copy(table_ref.at[sidx.at[j0 + 1]], rows1, sem1)
                pltpu.make_async_copy(table_ref.at[sidx.at[j0]], rows0,
                                      sem0).wait()
                pltpu.sync_copy(rows0, acc.at[didx.at[j0]], add=True)

                @pl.when(j0 + 2 < HCH)
                def _():
                    pltpu.async_copy(table_ref.at[sidx.at[j0 + 2]], rows0, sem0)

                pltpu.make_async_copy(table_ref.at[sidx.at[j0 + 1]], rows1,
                                      sem1).wait()
                pltpu.sync_copy(rows1, acc.at[didx.at[j0 + 1]], add=True)

        plsc.subcore_barrier()
        sl = pl.ds(r0, RPS)
        pltpu.sync_copy(acc.at[sl], out_ref.at[c, sl])

    return pl.kernel(
        body,
        out_type=jax.ShapeDtypeStruct((NC, NPAD, D), jnp.float32),
        mesh=_MESH,
        compiler_params=(pltpu.CompilerParams(use_tc_tiling_on_sc=False)
                         if D % 128 else None),
        scratch_types=[
            pltpu.VMEM((HCH, CHUNK), jnp.int32),
            pltpu.VMEM((HCH, CHUNK), jnp.int32),
            pltpu.VMEM((CHUNK, D), jnp.float32),
            pltpu.VMEM((CHUNK, D), jnp.float32),
            pltpu.VMEM_SHARED((NPAD, D), jnp.float32),
            pltpu.SemaphoreType.DMA,
            pltpu.SemaphoreType.DMA,
            pltpu.SemaphoreType.DMA,
        ],
    )(table, src2, dst2, zeros_nd)


def _sc_counts(src2, dst2, ones_c, zeros_c):
    def body(src_ref, dst_ref, ones_ref, zc_ref,
             dcnt_ref, bcnt_ref, sidx, didx, ones_v, dacc, bacc):
        c = lax.axis_index("c")
        s = lax.axis_index("s")
        w = c * NS + s
        r0 = s * RPS
        pltpu.sync_copy(zc_ref.at[pl.ds(r0, RPS)], dacc.at[pl.ds(r0, RPS)])
        pltpu.sync_copy(zc_ref.at[pl.ds(r0, RPS)], bacc.at[pl.ds(r0, RPS)])
        pltpu.sync_copy(ones_ref, ones_v)
        pltpu.sync_copy(src_ref.at[pl.ds(w * NCHUNK, NCHUNK)], sidx)
        pltpu.sync_copy(dst_ref.at[pl.ds(w * NCHUNK, NCHUNK)], didx)
        plsc.subcore_barrier()

        @pl.loop(0, NCHUNK)
        def _(j):
            pltpu.sync_copy(ones_v, dacc.at[sidx.at[j]], add=True)
            pltpu.sync_copy(ones_v, bacc.at[didx.at[j]], add=True)

        plsc.subcore_barrier()
        sl = pl.ds(r0, RPS)
        pltpu.sync_copy(dacc.at[sl], dcnt_ref.at[c, sl])
        pltpu.sync_copy(bacc.at[sl], bcnt_ref.at[c, sl])

    return pl.kernel(
        body,
        out_type=(jax.ShapeDtypeStruct((NC, NPAD, CW), jnp.float32),
                  jax.ShapeDtypeStruct((NC, NPAD, CW), jnp.float32)),
        mesh=_MESH,
        compiler_params=pltpu.CompilerParams(use_tc_tiling_on_sc=False),
        scratch_types=[
            pltpu.VMEM((NCHUNK, CHUNK), jnp.int32),
            pltpu.VMEM((NCHUNK, CHUNK), jnp.int32),
            pltpu.VMEM((CHUNK, CW), jnp.float32),
            pltpu.VMEM_SHARED((NPAD, CW), jnp.float32),
            pltpu.VMEM_SHARED((NPAD, CW), jnp.float32),
        ],
    )(src2, dst2, ones_c, zeros_c)


def _inv_from(cnt_ref):
    cnt = cnt_ref[0, :, 0:1] + cnt_ref[1, :, 0:1]
    return jnp.where(cnt > 0, 1.0 / jnp.where(cnt > 0, cnt, 1.0), 0.0)


def _mm_body(x_ref, w_ref, o_ref):
    o_ref[:N] = jnp.dot(x_ref[...], w_ref[...],
                        preferred_element_type=jnp.float32)
    o_ref[N:] = jnp.zeros((NPAD - N, o_ref.shape[1]), jnp.float32)


def _mm(x, wT):
    return pl.pallas_call(
        _mm_body,
        out_shape=jax.ShapeDtypeStruct((NPAD, wT.shape[1]), jnp.float32),
    )(x, wT)


def _combine_body(p_ref, cnt_ref, o_ref):
    o_ref[...] = (p_ref[0] + p_ref[1]) * _inv_from(cnt_ref)


def _combine(p, cnt):
    return pl.pallas_call(
        _combine_body,
        out_shape=jax.ShapeDtypeStruct(p.shape[1:], jnp.float32),
    )(p, cnt)


def _layer_body(p_ref, cnt_ref, b_ref, w_ref, h_ref, xw_ref):
    h = jnp.maximum((p_ref[0] + p_ref[1]) * _inv_from(cnt_ref) + b_ref[...],
                    0.0)[:N]
    h_ref[...] = h
    xw_ref[:N] = jnp.dot(h, w_ref[...], preferred_element_type=jnp.float32)
    xw_ref[N:] = jnp.zeros((NPAD - N, xw_ref.shape[1]), jnp.float32)


def _layer(p, cnt, b, wT):
    d = p.shape[2]
    return pl.pallas_call(
        _layer_body,
        out_shape=(jax.ShapeDtypeStruct((N, d), jnp.float32),
                   jax.ShapeDtypeStruct((NPAD, wT.shape[1]), jnp.float32)),
    )(p, cnt, b, wT)


def _final_body(p_ref, cnt_ref, b_ref, w_ref, bp_ref, h_ref, z_ref):
    h = jnp.maximum((p_ref[0] + p_ref[1]) * _inv_from(cnt_ref) + b_ref[...],
                    0.0)[:N]
    h_ref[...] = h
    z_ref[...] = jnp.dot(h, w_ref[...],
                         preferred_element_type=jnp.float32) + bp_ref[...]


def _final(p, cnt, b, wT, bp):
    d = p.shape[2]
    return pl.pallas_call(
        _final_body,
        out_shape=(jax.ShapeDtypeStruct((N, d), jnp.float32),
                   jax.ShapeDtypeStruct((N, wT.shape[1]), jnp.float32)),
    )(p, cnt, b, wT, bp)


def kernel(x, hyperedge_index, W1, b1, W2, b2, Wp, bp):
    pad_idx = N + (jnp.arange(NNZP - NNZ, dtype=jnp.int32) % (NPAD - N))
    node2 = jnp.concatenate([hyperedge_index[0], pad_idx]
                            ).reshape(NW * NCHUNK, CHUNK)
    edge2 = jnp.concatenate([hyperedge_index[1], pad_idx]
                            ).reshape(NW * NCHUNK, CHUNK)
    z128 = jnp.zeros((NPAD, D_HID), jnp.float32)
    z64 = jnp.zeros((NPAD, D_OUT), jnp.float32)
    zc = jnp.zeros((NPAD, CW), jnp.float32)
    ones_c = jnp.ones((CHUNK, CW), jnp.float32)

    dcnt, bcnt = _sc_counts(node2, edge2, ones_c, zc)
    xW1 = _mm(x, W1.T)
    e1p = _sc_pass(xW1, node2, edge2, z128)
    e1 = _combine(e1p, bcnt)
    h1p = _sc_pass(e1, edge2, node2, z128)
    h1f, xW2 = _layer(h1p, dcnt, b1.reshape(1, -1), W2.T)
    e2p = _sc_pass(xW2, node2, edge2, z64)
    e2 = _combine(e2p, bcnt)
    h2p = _sc_pass(e2, edge2, node2, z64)
    h2f, zf = _final(h2p, dcnt, b2.reshape(1, -1), Wp.T, bp.reshape(1, -1))
    return (zf, h1f, h2f)
